# baseline (speedup 1.0000x reference)
import functools

import jax
import jax.numpy as jnp
from jax import lax
from jax.experimental import pallas as pl
from jax.experimental.pallas import tpu as pltpu
from jax.experimental.pallas import tpu_sc as plsc

E = 8
K = 2
D = 1024
DFF = 4096
BLK = 256
DFF_T = 512
NF = DFF // DFF_T
LANES = 128
CH = 512

NC = 2
NS = 16
NW = NC * NS


def _router_body(x_ref, wg_ref, logits_ref, w01_ref, pos_ref, bexp_ref, c_ref):
    t = x_ref.shape[0]
    x = x_ref[...]
    wg = wg_ref[...]
    logits_full = lax.dot_general(
        x, wg, (((1,), (1,)), ((), ())), preferred_element_type=jnp.float32)
    logits_ref[...] = logits_full[:, :E]

    lane = lax.broadcasted_iota(jnp.int32, (t, LANES), 1).astype(jnp.float32)
    valid = lane < E
    lmask = jnp.where(valid, logits_full, -1e30)
    m = jnp.max(lmask, axis=1, keepdims=True)
    p = jnp.where(valid, jnp.exp(lmask - m), 0.0)
    prob = p / jnp.sum(p, axis=1, keepdims=True)

    m1 = jnp.max(prob, axis=1, keepdims=True)
    a1 = jnp.min(jnp.where((prob == m1) & valid, lane, float(LANES)),
                 axis=1, keepdims=True)
    prob2 = jnp.where(lane == a1, -1.0, prob)
    m2 = jnp.max(prob2, axis=1, keepdims=True)
    a2 = jnp.min(jnp.where((prob2 == m2) & valid, lane, float(LANES)),
                 axis=1, keepdims=True)
    wsum = m1 + m2
    w01_ref[:, 0:1] = m1 / wsum
    w01_ref[:, 1:2] = m2 / wsum

    c_mat = (lane == a1).astype(jnp.float32) + (lane == a2).astype(jnp.float32)
    c_ref[...] = c_mat

    cnt = jnp.sum(c_mat, axis=0, keepdims=True)
    cnt_i = cnt.astype(jnp.int32)
    pad_i = ((cnt_i + BLK - 1) // BLK) * BLK
    pad_f = pad_i.astype(jnp.float32)

    rr = lax.broadcasted_iota(jnp.int32, (LANES, LANES), 0).astype(jnp.float32)
    cc = lax.broadcasted_iota(jnp.int32, (LANES, LANES), 1).astype(jnp.float32)
    strict_upper = (rr < cc).astype(jnp.float32)
    off = lax.dot_general(pad_f, strict_upper, (((1,), (0,)), ((), ())),
                          preferred_element_type=jnp.float32)
    end = off + pad_f

    eye = (rr == cc).astype(jnp.float32)
    end_col = lax.dot_general(eye, end, (((1,), (1,)), ((), ())),
                              preferred_element_type=jnp.float32)
    endb_col = end_col / float(BLK)
    bidx = lax.broadcasted_iota(jnp.int32, (LANES, LANES), 1).astype(jnp.float32)
    ind = (endb_col <= bidx).astype(jnp.float32)
    ones_row = jnp.ones((1, LANES), jnp.float32)
    bexp = lax.dot_general(ones_row, ind, (((1,), (0,)), ((), ())),
                           preferred_element_type=jnp.float32)
    bexp = jnp.minimum(bexp, float(E - 1))
    nb = (t * K) // BLK + E
    used = jnp.sum(pad_f, axis=1, keepdims=True) / float(BLK)
    lane_r = lax.broadcasted_iota(jnp.int32, (1, LANES), 1)
    bexp_ref[...] = jnp.where(lane_r == nb, used, bexp).astype(jnp.int32)

    rs = lax.broadcasted_iota(jnp.int32, (CH, CH), 0).astype(jnp.float32)
    cs = lax.broadcasted_iota(jnp.int32, (CH, CH), 1).astype(jnp.float32)
    strict_lower = (cs < rs).astype(jnp.float32)
    run = jnp.zeros((1, LANES), jnp.float32)
    for i in range(t // CH):
        sl = pl.ds(i * CH, CH)
        c_chunk = c_ref[sl, :]
        p_chunk = lax.dot_general(
            strict_lower, c_chunk, (((1,), (0,)), ((), ())),
            preferred_element_type=jnp.float32) + run
        posmat = off + p_chunk
        a1_c = a1[i * CH:(i + 1) * CH, :]
        a2_c = a2[i * CH:(i + 1) * CH, :]
        lane_c = lax.broadcasted_iota(jnp.int32, (CH, LANES), 1).astype(jnp.float32)
        pos0 = jnp.sum(jnp.where(lane_c == a1_c, posmat, 0.0),
                       axis=1, keepdims=True)
        pos1 = jnp.sum(jnp.where(lane_c == a2_c, posmat, 0.0),
                       axis=1, keepdims=True)
        pos_ref[sl, 0:1] = pos0.astype(jnp.int32)
        pos_ref[sl, 1:2] = pos1.astype(jnp.int32)
        run = run + jnp.sum(c_chunk, axis=0, keepdims=True)


def _run_router(x, w_gate_pad):
    t = x.shape[0]
    return pl.pallas_call(
        _router_body,
        out_shape=[
            jax.ShapeDtypeStruct((t, E), jnp.float32),
            jax.ShapeDtypeStruct((t, K), jnp.float32),
            jax.ShapeDtypeStruct((t, K), jnp.int32),
            jax.ShapeDtypeStruct((1, LANES), jnp.int32),
        ],
        scratch_shapes=[pltpu.VMEM((t, LANES), jnp.float32)],
    )(x, w_gate_pad)


def _ffn_body(bexp_ref, xs_ref, wg_ref, wu_ref, wd_ref, o_ref, acc_ref):
    f = pl.program_id(0)
    b = pl.program_id(1)
    nb = pl.num_programs(1)
    used = bexp_ref[nb]

    @pl.when(b < used)
    def _():
        x = xs_ref[...].astype(jnp.bfloat16)
        g = lax.dot_general(x, wg_ref[0], (((1,), (1,)), ((), ())),
                            preferred_element_type=jnp.float32)
        u = lax.dot_general(x, wu_ref[0], (((1,), (1,)), ((), ())),
                            preferred_element_type=jnp.float32)
        h = (jax.nn.gelu(g, approximate=True) * u).astype(jnp.bfloat16)
        part = lax.dot_general(h, wd_ref[0], (((1,), (1,)), ((), ())),
                               preferred_element_type=jnp.float32)
        row = pl.ds(b * BLK, BLK)

        @pl.when(f == 0)
        def _():
            acc_ref[row, :] = part

        @pl.when(f > 0)
        def _():
            acc_ref[row, :] = acc_ref[row, :] + part

        @pl.when(f == NF - 1)
        def _():
            o_ref[...] = acc_ref[row, :]


def _run_ffn(bexp, xs, wg, wu, wd, nb):
    nslot = nb * BLK
    grid_spec = pltpu.PrefetchScalarGridSpec(
        num_scalar_prefetch=1,
        grid=(NF, nb),
        in_specs=[
            pl.BlockSpec((BLK, D), lambda f, b, be: (b, 0)),
            pl.BlockSpec((1, DFF_T, D), lambda f, b, be: (be[b], f, 0)),
            pl.BlockSpec((1, DFF_T, D), lambda f, b, be: (be[b], f, 0)),
            pl.BlockSpec((1, D, DFF_T), lambda f, b, be: (be[b], 0, f)),
        ],
        out_specs=pl.BlockSpec(
            (BLK, D), lambda f, b, be: (jnp.where(f == NF - 1, b, 0), 0)),
        scratch_shapes=[pltpu.VMEM((nslot, D), jnp.float32)],
    )
    return pl.pallas_call(
        _ffn_body,
        grid_spec=grid_spec,
        out_shape=jax.ShapeDtypeStruct((nslot, D), jnp.float32),
        compiler_params=pltpu.CompilerParams(
            dimension_semantics=("arbitrary", "arbitrary")),
    )(bexp, xs, wg, wu, wd)


def _make_dispatch(t, nslot):
    rows_per_w = t // NW
    chunk = 64
    mesh = plsc.VectorSubcoreMesh(core_axis_name="c", subcore_axis_name="s")

    @functools.partial(
        pl.kernel,
        out_type=jax.ShapeDtypeStruct((nslot, D), jnp.float32),
        mesh=mesh,
        scratch_types=[
            pltpu.VMEM((chunk, D), jnp.float32),
            pltpu.VMEM((chunk,), jnp.int32),
            pltpu.VMEM((chunk,), jnp.int32),
            pltpu.SemaphoreType.DMA,
        ],
    )
    def dispatch(x_hbm, pos0_hbm, pos1_hbm, xs_hbm, rows_v, i0_v, i1_v, sem):
        wid = lax.axis_index("s") * NC + lax.axis_index("c")
        for c in range(rows_per_w // chunk):
            base = wid * rows_per_w + c * chunk
            pltpu.sync_copy(x_hbm.at[pl.ds(base, chunk)], rows_v)
            pltpu.sync_copy(pos0_hbm.at[pl.ds(base, chunk)], i0_v)
            pltpu.sync_copy(pos1_hbm.at[pl.ds(base, chunk)], i1_v)
            cp0 = pltpu.async_copy(rows_v, xs_hbm.at[i0_v], sem)
            cp1 = pltpu.async_copy(rows_v, xs_hbm.at[i1_v], sem)
            cp0.wait()
            cp1.wait()

    return dispatch


def _make_combine(t, nslot):
    rows_per_w = t // NW
    chunk = 32
    nlc = D // 16
    mesh = plsc.VectorSubcoreMesh(core_axis_name="c", subcore_axis_name="s")

    @functools.partial(
        pl.kernel,
        out_type=jax.ShapeDtypeStruct((t, D), jnp.float32),
        mesh=mesh,
        scratch_types=[
            pltpu.VMEM((chunk, D), jnp.float32),
            pltpu.VMEM((chunk, D), jnp.float32),
            pltpu.VMEM((chunk, D), jnp.float32),
            pltpu.VMEM((chunk, 16), jnp.float32),
            pltpu.VMEM((chunk, 16), jnp.float32),
            pltpu.VMEM((chunk,), jnp.int32),
            pltpu.VMEM((chunk,), jnp.int32),
            pltpu.SemaphoreType.DMA,
        ],
    )
    def combine(y_hbm, pos0_hbm, pos1_hbm, wr0_hbm, wr1_hbm, out_hbm,
                b0_v, b1_v, o_v, w0_v, w1_v, i0_v, i1_v, sem):
        wid = lax.axis_index("s") * NC + lax.axis_index("c")
        for c in range(rows_per_w // chunk):
            base = wid * rows_per_w + c * chunk
            pltpu.sync_copy(pos0_hbm.at[pl.ds(base, chunk)], i0_v)
            pltpu.sync_copy(pos1_hbm.at[pl.ds(base, chunk)], i1_v)
            pltpu.sync_copy(wr0_hbm.at[pl.ds(base, chunk)], w0_v)
            pltpu.sync_copy(wr1_hbm.at[pl.ds(base, chunk)], w1_v)
            cp0 = pltpu.async_copy(y_hbm.at[i0_v], b0_v, sem)
            cp1 = pltpu.async_copy(y_hbm.at[i1_v], b1_v, sem)
            cp0.wait()
            cp1.wait()

            def row_body(r, carry):
                w0 = w0_v[r]
                w1 = w1_v[r]
                for cc in range(nlc):
                    sl = pl.ds(cc * 16, 16)
                    o_v[r, sl] = w0 * b0_v[r, sl] + w1 * b1_v[r, sl]
                return carry

            lax.fori_loop(0, chunk, row_body, 0)
            pltpu.sync_copy(o_v, out_hbm.at[pl.ds(base, chunk)])

    return combine


@jax.jit
def kernel(hidden_states, W_gate, Wg, Wu, Wd):
    b, s, d = hidden_states.shape
    t = b * s
    x = hidden_states.reshape(t, d)

    w_gate_pad = jnp.zeros((LANES, d), jnp.float32).at[:E].set(W_gate)
    logits, w01, pos, bexp2d = _run_router(x, w_gate_pad)

    nb = (t * K) // BLK + E
    nslot = nb * BLK
    bexp = bexp2d[0, :nb + 1]
    pos0 = pos[:, 0]
    pos1 = pos[:, 1]
    wr0 = jnp.broadcast_to(w01[:, 0:1], (t, 16))
    wr1 = jnp.broadcast_to(w01[:, 1:2], (t, 16))

    xs = _make_dispatch(t, nslot)(x, pos0, pos1)
    y = _run_ffn(bexp, xs, Wg.astype(jnp.bfloat16), Wu.astype(jnp.bfloat16),
                 Wd.astype(jnp.bfloat16), nb)
    out = _make_combine(t, nslot)(y, pos0, pos1, wr0, wr1)
    return out.reshape(b, s, d), logits


# speedup vs baseline: 1.1739x; 1.1739x over previous
import functools

import jax
import jax.numpy as jnp
from jax import lax
from jax.experimental import pallas as pl
from jax.experimental.pallas import tpu as pltpu
from jax.experimental.pallas import tpu_sc as plsc

E = 8
K = 2
D = 1024
DFF = 4096
BLK = 256
DFF_T = 512
NF = DFF // DFF_T
LANES = 128
CH = 512

NC = 2
NS = 16
NW = NC * NS


def _router_body(x_ref, wg_ref, logits_ref, w01_ref, pos_ref, bexp_ref, c_ref):
    t = x_ref.shape[0]
    x = x_ref[...]
    wg = wg_ref[...]
    logits_full = lax.dot_general(
        x, wg, (((1,), (1,)), ((), ())), preferred_element_type=jnp.float32)
    logits_ref[...] = logits_full[:, :E]

    lane = lax.broadcasted_iota(jnp.int32, (t, LANES), 1).astype(jnp.float32)
    valid = lane < E
    lmask = jnp.where(valid, logits_full, -1e30)
    m = jnp.max(lmask, axis=1, keepdims=True)
    p = jnp.where(valid, jnp.exp(lmask - m), 0.0)
    prob = p / jnp.sum(p, axis=1, keepdims=True)

    m1 = jnp.max(prob, axis=1, keepdims=True)
    a1 = jnp.min(jnp.where((prob == m1) & valid, lane, float(LANES)),
                 axis=1, keepdims=True)
    prob2 = jnp.where(lane == a1, -1.0, prob)
    m2 = jnp.max(prob2, axis=1, keepdims=True)
    a2 = jnp.min(jnp.where((prob2 == m2) & valid, lane, float(LANES)),
                 axis=1, keepdims=True)
    wsum = m1 + m2
    w01_ref[:, 0:1] = m1 / wsum
    w01_ref[:, 1:2] = m2 / wsum

    c_mat = (lane == a1).astype(jnp.float32) + (lane == a2).astype(jnp.float32)
    c_ref[...] = c_mat

    cnt = jnp.sum(c_mat, axis=0, keepdims=True)
    cnt_i = cnt.astype(jnp.int32)
    pad_i = ((cnt_i + BLK - 1) // BLK) * BLK
    pad_f = pad_i.astype(jnp.float32)

    rr = lax.broadcasted_iota(jnp.int32, (LANES, LANES), 0).astype(jnp.float32)
    cc = lax.broadcasted_iota(jnp.int32, (LANES, LANES), 1).astype(jnp.float32)
    strict_upper = (rr < cc).astype(jnp.float32)
    off = lax.dot_general(pad_f, strict_upper, (((1,), (0,)), ((), ())),
                          preferred_element_type=jnp.float32)
    end = off + pad_f

    eye = (rr == cc).astype(jnp.float32)
    end_col = lax.dot_general(eye, end, (((1,), (1,)), ((), ())),
                              preferred_element_type=jnp.float32)
    endb_col = end_col / float(BLK)
    bidx = lax.broadcasted_iota(jnp.int32, (LANES, LANES), 1).astype(jnp.float32)
    ind = (endb_col <= bidx).astype(jnp.float32)
    ones_row = jnp.ones((1, LANES), jnp.float32)
    bexp = lax.dot_general(ones_row, ind, (((1,), (0,)), ((), ())),
                           preferred_element_type=jnp.float32)
    bexp = jnp.minimum(bexp, float(E - 1))
    nb = (t * K) // BLK + E
    used = jnp.sum(pad_f, axis=1, keepdims=True) / float(BLK)
    lane_r = lax.broadcasted_iota(jnp.int32, (1, LANES), 1)
    bexp_ref[...] = jnp.where(lane_r == nb, used, bexp).astype(jnp.int32)

    rs = lax.broadcasted_iota(jnp.int32, (CH, CH), 0).astype(jnp.float32)
    cs = lax.broadcasted_iota(jnp.int32, (CH, CH), 1).astype(jnp.float32)
    strict_lower = (cs < rs).astype(jnp.float32)
    run = jnp.zeros((1, LANES), jnp.float32)
    for i in range(t // CH):
        sl = pl.ds(i * CH, CH)
        c_chunk = c_ref[sl, :]
        p_chunk = lax.dot_general(
            strict_lower, c_chunk, (((1,), (0,)), ((), ())),
            preferred_element_type=jnp.float32) + run
        posmat = off + p_chunk
        a1_c = a1[i * CH:(i + 1) * CH, :]
        a2_c = a2[i * CH:(i + 1) * CH, :]
        lane_c = lax.broadcasted_iota(jnp.int32, (CH, LANES), 1).astype(jnp.float32)
        pos0 = jnp.sum(jnp.where(lane_c == a1_c, posmat, 0.0),
                       axis=1, keepdims=True)
        pos1 = jnp.sum(jnp.where(lane_c == a2_c, posmat, 0.0),
                       axis=1, keepdims=True)
        pos_ref[sl, 0:1] = pos0.astype(jnp.int32)
        pos_ref[sl, 1:2] = pos1.astype(jnp.int32)
        run = run + jnp.sum(c_chunk, axis=0, keepdims=True)


def _run_router(x, w_gate_pad):
    t = x.shape[0]
    return pl.pallas_call(
        _router_body,
        out_shape=[
            jax.ShapeDtypeStruct((t, E), jnp.float32),
            jax.ShapeDtypeStruct((t, K), jnp.float32),
            jax.ShapeDtypeStruct((t, K), jnp.int32),
            jax.ShapeDtypeStruct((1, LANES), jnp.int32),
        ],
        scratch_shapes=[pltpu.VMEM((t, LANES), jnp.float32)],
    )(x, w_gate_pad)


def _ffn_body(bexp_ref, xs_ref, wg_ref, wu_ref, wd_ref, o_ref, acc_ref):
    f = pl.program_id(0)
    b = pl.program_id(1)
    nb = pl.num_programs(1)
    used = bexp_ref[nb]

    @pl.when(b < used)
    def _():
        x = xs_ref[...]
        g = lax.dot_general(x, wg_ref[0], (((1,), (1,)), ((), ())),
                            preferred_element_type=jnp.float32)
        u = lax.dot_general(x, wu_ref[0], (((1,), (1,)), ((), ())),
                            preferred_element_type=jnp.float32)
        h = jax.nn.gelu(g, approximate=True) * u
        part = lax.dot_general(h, wd_ref[0], (((1,), (1,)), ((), ())),
                               preferred_element_type=jnp.float32)
        row = pl.ds(b * BLK, BLK)

        @pl.when(f == 0)
        def _():
            acc_ref[row, :] = part

        @pl.when(f > 0)
        def _():
            acc_ref[row, :] = acc_ref[row, :] + part

        @pl.when(f == NF - 1)
        def _():
            o_ref[...] = acc_ref[row, :]


def _run_ffn(bexp, xs, wg, wu, wd, nb):
    nslot = nb * BLK
    grid_spec = pltpu.PrefetchScalarGridSpec(
        num_scalar_prefetch=1,
        grid=(NF, nb),
        in_specs=[
            pl.BlockSpec((BLK, D), lambda f, b, be: (b, 0)),
            pl.BlockSpec((1, DFF_T, D), lambda f, b, be: (be[b], f, 0)),
            pl.BlockSpec((1, DFF_T, D), lambda f, b, be: (be[b], f, 0)),
            pl.BlockSpec((1, D, DFF_T), lambda f, b, be: (be[b], 0, f)),
        ],
        out_specs=pl.BlockSpec(
            (BLK, D), lambda f, b, be: (jnp.where(f == NF - 1, b, 0), 0)),
        scratch_shapes=[pltpu.VMEM((nslot, D), jnp.float32)],
    )
    return pl.pallas_call(
        _ffn_body,
        grid_spec=grid_spec,
        out_shape=jax.ShapeDtypeStruct((nslot, D), jnp.float32),
        compiler_params=pltpu.CompilerParams(
            dimension_semantics=("arbitrary", "arbitrary")),
    )(bexp, xs, wg, wu, wd)


def _make_dispatch(t, nslot):
    rows_per_w = t // NW
    chunk = 64
    mesh = plsc.VectorSubcoreMesh(core_axis_name="c", subcore_axis_name="s")

    @functools.partial(
        pl.kernel,
        out_type=jax.ShapeDtypeStruct((nslot, D), jnp.float32),
        mesh=mesh,
        scratch_types=[
            pltpu.VMEM((chunk, D), jnp.float32),
            pltpu.VMEM((chunk,), jnp.int32),
            pltpu.VMEM((chunk,), jnp.int32),
            pltpu.SemaphoreType.DMA,
        ],
    )
    def dispatch(x_hbm, pos0_hbm, pos1_hbm, xs_hbm, rows_v, i0_v, i1_v, sem):
        wid = lax.axis_index("s") * NC + lax.axis_index("c")
        for c in range(rows_per_w // chunk):
            base = wid * rows_per_w + c * chunk
            pltpu.sync_copy(x_hbm.at[pl.ds(base, chunk)], rows_v)
            pltpu.sync_copy(pos0_hbm.at[pl.ds(base, chunk)], i0_v)
            pltpu.sync_copy(pos1_hbm.at[pl.ds(base, chunk)], i1_v)
            cp0 = pltpu.async_copy(rows_v, xs_hbm.at[i0_v], sem)
            cp1 = pltpu.async_copy(rows_v, xs_hbm.at[i1_v], sem)
            cp0.wait()
            cp1.wait()

    return dispatch


def _make_combine(t, nslot):
    rows_per_w = t // NW
    chunk = 32
    nlc = D // 16
    mesh = plsc.VectorSubcoreMesh(core_axis_name="c", subcore_axis_name="s")

    @functools.partial(
        pl.kernel,
        out_type=jax.ShapeDtypeStruct((t, D), jnp.float32),
        mesh=mesh,
        scratch_types=[
            pltpu.VMEM((chunk, D), jnp.float32),
            pltpu.VMEM((chunk, D), jnp.float32),
            pltpu.VMEM((chunk, D), jnp.float32),
            pltpu.VMEM((chunk, 16), jnp.float32),
            pltpu.VMEM((chunk, 16), jnp.float32),
            pltpu.VMEM((chunk,), jnp.int32),
            pltpu.VMEM((chunk,), jnp.int32),
            pltpu.SemaphoreType.DMA,
        ],
    )
    def combine(y_hbm, pos0_hbm, pos1_hbm, wr0_hbm, wr1_hbm, out_hbm,
                b0_v, b1_v, o_v, w0_v, w1_v, i0_v, i1_v, sem):
        wid = lax.axis_index("s") * NC + lax.axis_index("c")
        for c in range(rows_per_w // chunk):
            base = wid * rows_per_w + c * chunk
            pltpu.sync_copy(pos0_hbm.at[pl.ds(base, chunk)], i0_v)
            pltpu.sync_copy(pos1_hbm.at[pl.ds(base, chunk)], i1_v)
            pltpu.sync_copy(wr0_hbm.at[pl.ds(base, chunk)], w0_v)
            pltpu.sync_copy(wr1_hbm.at[pl.ds(base, chunk)], w1_v)
            cp0 = pltpu.async_copy(y_hbm.at[i0_v], b0_v, sem)
            cp1 = pltpu.async_copy(y_hbm.at[i1_v], b1_v, sem)
            cp0.wait()
            cp1.wait()

            def row_body(r, carry):
                w0 = w0_v[r]
                w1 = w1_v[r]
                for cc in range(nlc):
                    sl = pl.ds(cc * 16, 16)
                    o_v[r, sl] = w0 * b0_v[r, sl] + w1 * b1_v[r, sl]
                return carry

            lax.fori_loop(0, chunk, row_body, 0)
            pltpu.sync_copy(o_v, out_hbm.at[pl.ds(base, chunk)])

    return combine


@jax.jit
def kernel(hidden_states, W_gate, Wg, Wu, Wd):
    b, s, d = hidden_states.shape
    t = b * s
    x = hidden_states.reshape(t, d)

    w_gate_pad = jnp.zeros((LANES, d), jnp.float32).at[:E].set(W_gate)
    logits, w01, pos, bexp2d = _run_router(x, w_gate_pad)

    nb = (t * K) // BLK + E
    nslot = nb * BLK
    bexp = bexp2d[0, :nb + 1]
    pos0 = pos[:, 0]
    pos1 = pos[:, 1]
    wr0 = jnp.broadcast_to(w01[:, 0:1], (t, 16))
    wr1 = jnp.broadcast_to(w01[:, 1:2], (t, 16))

    xs = _make_dispatch(t, nslot)(x, pos0, pos1)
    y = _run_ffn(bexp, xs, Wg, Wu, Wd, nb)
    out = _make_combine(t, nslot)(y, pos0, pos1, wr0, wr1)
    return out.reshape(b, s, d), logits
